# Initial kernel scaffold; baseline (speedup 1.0000x reference)
import functools

import jax
import jax.numpy as jnp
from jax import lax
from jax.experimental import pallas as pl
from jax.experimental.pallas import tpu as pltpu
from jax.experimental.pallas import tpu_sc as plsc

NC = 2
NS = 16
NW = NC * NS


def _sc_aggregate(x, src, dst, n, d):
  e = src.shape[0]
  assert e % NW == 0
  e_per_w = e // NW
  C = 80
  assert e_per_w % C == 0
  nchunks = e_per_w // C
  assert n % NS == 0
  rows_per_sub = n // NS

  zeros_feat = jnp.zeros((rows_per_sub, d), jnp.float32)
  zeros_cnt = jnp.zeros((rows_per_sub, 16), jnp.float32)
  ones_blk = jnp.ones((C, 16), jnp.float32)

  mesh = plsc.VectorSubcoreMesh(core_axis_name="c", subcore_axis_name="s",
                                num_cores=NC, num_subcores=NS)

  def body(x_hbm, src_hbm, dst_hbm, zf_hbm, zc_hbm, ones_hbm,
           sum_out, cnt_out, acc, cnt, src_v, dst_v, rows_v, ones_v, sem):
    cid = lax.axis_index("c")
    sid = lax.axis_index("s")
    wid = cid * NS + sid
    base_n = sid * rows_per_sub
    pltpu.sync_copy(zf_hbm, acc.at[pl.ds(base_n, rows_per_sub)])
    pltpu.sync_copy(zc_hbm, cnt.at[pl.ds(base_n, rows_per_sub)])
    pltpu.sync_copy(ones_hbm, ones_v)
    plsc.subcore_barrier()

    base_e = wid * e_per_w

    def chunk(i, carry):
      off = base_e + i * C
      pltpu.sync_copy(src_hbm.at[pl.ds(off, C)], src_v)
      pltpu.sync_copy(dst_hbm.at[pl.ds(off, C)], dst_v)
      pltpu.async_copy(x_hbm.at[src_v], rows_v, sem).wait()
      pltpu.sync_copy(rows_v, acc.at[dst_v], add=True)
      pltpu.sync_copy(ones_v, cnt.at[dst_v], add=True)
      return carry

    lax.fori_loop(0, nchunks, chunk, 0)
    plsc.subcore_barrier()
    pltpu.sync_copy(acc.at[pl.ds(base_n, rows_per_sub)],
                    sum_out.at[cid, pl.ds(base_n, rows_per_sub)])
    pltpu.sync_copy(cnt.at[pl.ds(base_n, rows_per_sub)],
                    cnt_out.at[cid, pl.ds(base_n, rows_per_sub)])

  call = pl.kernel(
      body,
      out_type=(
          jax.ShapeDtypeStruct((NC, n, d), jnp.float32),
          jax.ShapeDtypeStruct((NC, n, 16), jnp.float32),
      ),
      mesh=mesh,
      scratch_types=[
          pltpu.VMEM_SHARED((n, d), jnp.float32),
          pltpu.VMEM_SHARED((n, 16), jnp.float32),
          pltpu.VMEM((C,), jnp.int32),
          pltpu.VMEM((C,), jnp.int32),
          pltpu.VMEM((C, d), jnp.float32),
          pltpu.VMEM((C, 16), jnp.float32),
          pltpu.SemaphoreType.DMA,
      ],
  )
  return call(x, src, dst, zeros_feat, zeros_cnt, ones_blk)


def _tc_mlp(x, s0, s1, c0, c1, W_lT, W_rT, W1T, W2T, W3T, b_l, b1, b2, b3):
  n, d = x.shape
  out_dim = W3T.shape[1]
  R = 1000
  assert n % R == 0
  grid = n // R

  def body(xb, s0b, s1b, c0b, c1b, wl, wr, w1, w2, w3,
           bl, bb1, bb2, bb3, ob):
    counts = c0b[:, :1] + c1b[:, :1]
    summed = s0b[...] + s1b[...]
    mean = summed / jnp.maximum(counts, 1.0)
    h = (jnp.dot(mean, wl[...], preferred_element_type=jnp.float32)
         + jnp.dot(xb[...], wr[...], preferred_element_type=jnp.float32)
         + bl[...])
    h1 = jnp.maximum(
        jnp.dot(h, w1[...], preferred_element_type=jnp.float32) + bb1[...], 0.0)
    h2 = jnp.maximum(
        jnp.dot(h1, w2[...], preferred_element_type=jnp.float32) + bb2[...], 0.0)
    ob[...] = jnp.dot(h2, w3[...], preferred_element_type=jnp.float32) + bb3[...]

  row_spec = lambda c: pl.BlockSpec((R, c), lambda i: (i, 0))
  full_spec = lambda r, c: pl.BlockSpec((r, c), lambda i: (0, 0))
  return pl.pallas_call(
      body,
      grid=(grid,),
      in_specs=[
          row_spec(d), row_spec(d), row_spec(d), row_spec(16), row_spec(16),
          full_spec(*W_lT.shape), full_spec(*W_rT.shape),
          full_spec(*W1T.shape), full_spec(*W2T.shape), full_spec(*W3T.shape),
          full_spec(*b_l.shape), full_spec(*b1.shape),
          full_spec(*b2.shape), full_spec(*b3.shape),
      ],
      out_specs=row_spec(out_dim),
      out_shape=jax.ShapeDtypeStruct((n, out_dim), jnp.float32),
  )(x, s0, s1, c0, c1, W_lT, W_rT, W1T, W2T, W3T, b_l, b1, b2, b3)


@jax.jit
def kernel(x, edge_index, W_l, b_l, W_r, W1, b1, W2, b2, W3, b3):
  n, d = x.shape
  src = edge_index[0]
  dst = edge_index[1]
  sums, counts16 = _sc_aggregate(x, src, dst, n, d)
  return _tc_mlp(
      x, sums[0], sums[1], counts16[0], counts16[1],
      W_l.T, W_r.T, W1.T, W2.T, W3.T,
      b_l.reshape(1, -1), b1.reshape(1, -1), b2.reshape(1, -1),
      b3.reshape(1, -1))


# speedup vs baseline: 5.5035x; 5.5035x over previous
import jax
import jax.numpy as jnp
from jax import lax
from jax.experimental import pallas as pl
from jax.experimental.pallas import tpu as pltpu
from jax.experimental.pallas import tpu_sc as plsc

NC = 2
NS = 16
NW = NC * NS


def _sc_aggregate(x, src, dst, n_pad, n_cnt):
  e = src.shape[0]
  d = x.shape[1]
  assert e % NW == 0
  e_per_w = e // NW
  C = 80
  assert e_per_w % C == 0
  nchunks = e_per_w // C
  rows_per_sub = n_pad // NS
  cnt_per_sub = n_cnt // NS

  zeros_blk = jnp.zeros((rows_per_sub, d), jnp.float32)
  zeros_cnt = jnp.zeros((cnt_per_sub,), jnp.float32)
  ones_blk = jnp.ones((C,), jnp.float32)

  mesh = plsc.VectorSubcoreMesh(core_axis_name="c", subcore_axis_name="s",
                                num_cores=NC, num_subcores=NS)

  def body(x_hbm, src_hbm, dst_hbm, zf_hbm, zc_hbm, ones_hbm,
           sum_out, cnt0_out, cnt1_out,
           acc, cnt_sh, src_v, dst_v, rows_v, ones_v, sem):
    cid = lax.axis_index("c")
    sid = lax.axis_index("s")
    wid = cid * NS + sid
    base_n = sid * rows_per_sub
    pltpu.sync_copy(zf_hbm, acc.at[pl.ds(base_n, rows_per_sub)])
    pltpu.sync_copy(zc_hbm, cnt_sh.at[pl.ds(sid * cnt_per_sub, cnt_per_sub)])
    pltpu.sync_copy(ones_hbm, ones_v)
    plsc.subcore_barrier()

    base_e = wid * e_per_w

    def chunk(i, carry):
      off = base_e + i * C
      pltpu.sync_copy(src_hbm.at[pl.ds(off, C)], src_v)
      pltpu.sync_copy(dst_hbm.at[pl.ds(off, C)], dst_v)
      pltpu.async_copy(x_hbm.at[src_v], rows_v, sem).wait()
      pltpu.sync_copy(rows_v, acc.at[dst_v], add=True)
      pltpu.sync_copy(ones_v, cnt_sh.at[dst_v], add=True)
      return carry

    lax.fori_loop(0, nchunks, chunk, 0)
    plsc.subcore_barrier()
    pltpu.sync_copy(acc.at[pl.ds(base_n, rows_per_sub)],
                    sum_out.at[cid, pl.ds(base_n, rows_per_sub)])

    @pl.when(cid == 0)
    def _():
      pltpu.sync_copy(cnt_sh.at[pl.ds(sid * cnt_per_sub, cnt_per_sub)],
                      cnt0_out.at[pl.ds(sid * cnt_per_sub, cnt_per_sub)])

    @pl.when(cid == 1)
    def _():
      pltpu.sync_copy(cnt_sh.at[pl.ds(sid * cnt_per_sub, cnt_per_sub)],
                      cnt1_out.at[pl.ds(sid * cnt_per_sub, cnt_per_sub)])

  call = pl.kernel(
      body,
      out_type=(
          jax.ShapeDtypeStruct((NC, n_pad, d), jnp.float32),
          jax.ShapeDtypeStruct((n_cnt,), jnp.float32),
          jax.ShapeDtypeStruct((n_cnt,), jnp.float32),
      ),
      mesh=mesh,
      scratch_types=[
          pltpu.VMEM_SHARED((n_pad, d), jnp.float32),
          pltpu.VMEM_SHARED((n_cnt,), jnp.float32),
          pltpu.VMEM((C,), jnp.int32),
          pltpu.VMEM((C,), jnp.int32),
          pltpu.VMEM((C, d), jnp.float32),
          pltpu.VMEM((C,), jnp.float32),
          pltpu.SemaphoreType.DMA,
      ],
      compiler_params=pltpu.CompilerParams(use_tc_tiling_on_sc=False),
  )
  return call(x, src, dst, zeros_blk, zeros_cnt, ones_blk)


def _tc_mlp(x, s0, s1, counts, W_lT, W_rT, W1T, W2T, W3T, b_l, b1, b2, b3):
  n, d = x.shape
  out_dim = W3T.shape[1]
  R = 1000
  assert n % R == 0
  grid = n // R

  def body(xb, s0b, s1b, cb, wl, wr, w1, w2, w3, bl, bb1, bb2, bb3, ob):
    summed = s0b[...] + s1b[...]
    mean = summed / jnp.maximum(cb[...], 1.0)
    h = (jnp.dot(mean, wl[...], preferred_element_type=jnp.float32)
         + jnp.dot(xb[...], wr[...], preferred_element_type=jnp.float32)
         + bl[...])
    h1 = jnp.maximum(
        jnp.dot(h, w1[...], preferred_element_type=jnp.float32) + bb1[...], 0.0)
    h2 = jnp.maximum(
        jnp.dot(h1, w2[...], preferred_element_type=jnp.float32) + bb2[...], 0.0)
    ob[...] = jnp.dot(h2, w3[...], preferred_element_type=jnp.float32) + bb3[...]

  row_spec = lambda c: pl.BlockSpec((R, c), lambda i: (i, 0))
  full_spec = lambda r, c: pl.BlockSpec((r, c), lambda i: (0, 0))
  return pl.pallas_call(
      body,
      grid=(grid,),
      in_specs=[
          row_spec(d), row_spec(d), row_spec(d), row_spec(1),
          full_spec(*W_lT.shape), full_spec(*W_rT.shape),
          full_spec(*W1T.shape), full_spec(*W2T.shape), full_spec(*W3T.shape),
          full_spec(*b_l.shape), full_spec(*b1.shape),
          full_spec(*b2.shape), full_spec(*b3.shape),
      ],
      out_specs=row_spec(out_dim),
      out_shape=jax.ShapeDtypeStruct((n, out_dim), jnp.float32),
  )(x, s0, s1, counts, W_lT, W_rT, W1T, W2T, W3T, b_l, b1, b2, b3)


@jax.jit
def kernel(x, edge_index, W_l, b_l, W_r, W1, b1, W2, b2, W3, b3):
  n, d = x.shape
  src = edge_index[0]
  dst = edge_index[1]
  n_pad = ((n + NS * 8 - 1) // (NS * 8)) * (NS * 8)
  n_cnt = ((n + NS * 8 - 1) // (NS * 8)) * (NS * 8)
  sums, cnt0, cnt1 = _sc_aggregate(x, src, dst, n_pad, n_cnt)
  counts = (cnt0 + cnt1)[:n].reshape(n, 1)
  return _tc_mlp(
      x, sums[0], sums[1], counts,
      W_l.T, W_r.T, W1.T, W2.T, W3.T,
      b_l.reshape(1, -1), b1.reshape(1, -1), b2.reshape(1, -1),
      b3.reshape(1, -1))


# speedup vs baseline: 8.4260x; 1.5310x over previous
import jax
import jax.numpy as jnp
from jax import lax
from jax.experimental import pallas as pl
from jax.experimental.pallas import tpu as pltpu
from jax.experimental.pallas import tpu_sc as plsc

NC = 2
NS = 16
NW = NC * NS


def _sc_aggregate(x, src, dst, n_pad, n_cnt):
  e = src.shape[0]
  d = x.shape[1]
  assert e % NW == 0
  e_per_w = e // NW
  C = 80
  assert e_per_w % C == 0
  nchunks = e_per_w // C
  rows_per_sub = n_pad // NS
  cnt_per_sub = n_cnt // NS

  zeros_blk = jnp.zeros((rows_per_sub, d), jnp.float32)
  zeros_cnt = jnp.zeros((cnt_per_sub,), jnp.float32)
  ones_blk = jnp.ones((C,), jnp.float32)

  mesh = plsc.VectorSubcoreMesh(core_axis_name="c", subcore_axis_name="s",
                                num_cores=NC, num_subcores=NS)

  assert nchunks >= 3 and (nchunks - 3) % 2 == 0
  loop_iters = (nchunks - 3) // 2

  def body(x_hbm, src_hbm, dst_hbm, zf_hbm, zc_hbm, ones_hbm,
           sum_out, cnt0_out, cnt1_out,
           acc, cnt_sh, src0, src1, dst0, dst1, rows0, rows1, ones_v,
           g0, g1, s0, s1):
    cid = lax.axis_index("c")
    sid = lax.axis_index("s")
    wid = cid * NS + sid
    base_n = sid * rows_per_sub
    base_e = wid * e_per_w
    bufs = ((src0, dst0, rows0, g0, s0), (src1, dst1, rows1, g1, s1))

    def start_chunk(i, b):
      sv, dv, rv, g, _ = bufs[b]
      off = base_e + i * C
      pltpu.sync_copy(src_hbm.at[pl.ds(off, C)], sv)
      pltpu.sync_copy(dst_hbm.at[pl.ds(off, C)], dv)
      pltpu.async_copy(x_hbm.at[sv], rv, g)

    def wait_gather(b):
      sv, _, rv, g, _ = bufs[b]
      pltpu.make_async_copy(x_hbm.at[sv], rv, g).wait()

    def start_scatter(b):
      _, dv, rv, _, s = bufs[b]
      pltpu.async_copy(rv, acc.at[dv], s, add=True)
      pltpu.async_copy(ones_v, cnt_sh.at[dv], s, add=True)

    def wait_scatter(b):
      _, dv, rv, _, s = bufs[b]
      pltpu.make_async_copy(rv, acc.at[dv], s).wait()
      pltpu.make_async_copy(ones_v, cnt_sh.at[dv], s).wait()

    pltpu.sync_copy(ones_hbm, ones_v)
    start_chunk(0, 0)
    start_chunk(1, 1)
    pltpu.sync_copy(zf_hbm, acc.at[pl.ds(base_n, rows_per_sub)])
    pltpu.sync_copy(zc_hbm, cnt_sh.at[pl.ds(sid * cnt_per_sub, cnt_per_sub)])
    plsc.subcore_barrier()
    wait_gather(0)
    start_scatter(0)

    def chunk2(k, carry):
      i0 = 2 + 2 * k
      wait_scatter(0)
      start_chunk(i0, 0)
      wait_gather(1)
      start_scatter(1)
      wait_scatter(1)
      start_chunk(i0 + 1, 1)
      wait_gather(0)
      start_scatter(0)
      return carry

    lax.fori_loop(0, loop_iters, chunk2, 0)
    wait_scatter(0)
    start_chunk(nchunks - 1, 0)
    wait_gather(1)
    start_scatter(1)
    wait_gather(0)
    start_scatter(0)
    wait_scatter(1)
    wait_scatter(0)
    plsc.subcore_barrier()
    pltpu.sync_copy(acc.at[pl.ds(base_n, rows_per_sub)],
                    sum_out.at[cid, pl.ds(base_n, rows_per_sub)])

    @pl.when(cid == 0)
    def _():
      pltpu.sync_copy(cnt_sh.at[pl.ds(sid * cnt_per_sub, cnt_per_sub)],
                      cnt0_out.at[pl.ds(sid * cnt_per_sub, cnt_per_sub)])

    @pl.when(cid == 1)
    def _():
      pltpu.sync_copy(cnt_sh.at[pl.ds(sid * cnt_per_sub, cnt_per_sub)],
                      cnt1_out.at[pl.ds(sid * cnt_per_sub, cnt_per_sub)])

  call = pl.kernel(
      body,
      out_type=(
          jax.ShapeDtypeStruct((NC, n_pad, d), jnp.float32),
          jax.ShapeDtypeStruct((n_cnt,), jnp.float32),
          jax.ShapeDtypeStruct((n_cnt,), jnp.float32),
      ),
      mesh=mesh,
      scratch_types=[
          pltpu.VMEM_SHARED((n_pad, d), jnp.float32),
          pltpu.VMEM_SHARED((n_cnt,), jnp.float32),
          pltpu.VMEM((C,), jnp.int32),
          pltpu.VMEM((C,), jnp.int32),
          pltpu.VMEM((C,), jnp.int32),
          pltpu.VMEM((C,), jnp.int32),
          pltpu.VMEM((C, d), jnp.float32),
          pltpu.VMEM((C, d), jnp.float32),
          pltpu.VMEM((C,), jnp.float32),
          pltpu.SemaphoreType.DMA,
          pltpu.SemaphoreType.DMA,
          pltpu.SemaphoreType.DMA,
          pltpu.SemaphoreType.DMA,
      ],
      compiler_params=pltpu.CompilerParams(use_tc_tiling_on_sc=False),
  )
  return call(x, src, dst, zeros_blk, zeros_cnt, ones_blk)


def _tc_mlp(x, s0, s1, counts, W_lT, W_rT, W1T, W2T, W3T, b_l, b1, b2, b3):
  n, d = x.shape
  out_dim = W3T.shape[1]
  R = 1000
  assert n % R == 0
  grid = n // R

  def body(xb, s0b, s1b, cb, wl, wr, w1, w2, w3, bl, bb1, bb2, bb3, ob):
    summed = s0b[...] + s1b[...]
    mean = summed / jnp.maximum(cb[...], 1.0)
    h = (jnp.dot(mean, wl[...], preferred_element_type=jnp.float32)
         + jnp.dot(xb[...], wr[...], preferred_element_type=jnp.float32)
         + bl[...])
    h1 = jnp.maximum(
        jnp.dot(h, w1[...], preferred_element_type=jnp.float32) + bb1[...], 0.0)
    h2 = jnp.maximum(
        jnp.dot(h1, w2[...], preferred_element_type=jnp.float32) + bb2[...], 0.0)
    ob[...] = jnp.dot(h2, w3[...], preferred_element_type=jnp.float32) + bb3[...]

  row_spec = lambda c: pl.BlockSpec((R, c), lambda i: (i, 0))
  full_spec = lambda r, c: pl.BlockSpec((r, c), lambda i: (0, 0))
  return pl.pallas_call(
      body,
      grid=(grid,),
      in_specs=[
          row_spec(d), row_spec(d), row_spec(d), row_spec(1),
          full_spec(*W_lT.shape), full_spec(*W_rT.shape),
          full_spec(*W1T.shape), full_spec(*W2T.shape), full_spec(*W3T.shape),
          full_spec(*b_l.shape), full_spec(*b1.shape),
          full_spec(*b2.shape), full_spec(*b3.shape),
      ],
      out_specs=row_spec(out_dim),
      out_shape=jax.ShapeDtypeStruct((n, out_dim), jnp.float32),
  )(x, s0, s1, counts, W_lT, W_rT, W1T, W2T, W3T, b_l, b1, b2, b3)


@jax.jit
def kernel(x, edge_index, W_l, b_l, W_r, W1, b1, W2, b2, W3, b3):
  n, d = x.shape
  src = edge_index[0]
  dst = edge_index[1]
  n_pad = ((n + NS * 8 - 1) // (NS * 8)) * (NS * 8)
  n_cnt = ((n + NS * 8 - 1) // (NS * 8)) * (NS * 8)
  sums, cnt0, cnt1 = _sc_aggregate(x, src, dst, n_pad, n_cnt)
  counts = (cnt0 + cnt1)[:n].reshape(n, 1)
  return _tc_mlp(
      x, sums[0], sums[1], counts,
      W_l.T, W_r.T, W1.T, W2.T, W3.T,
      b_l.reshape(1, -1), b1.reshape(1, -1), b2.reshape(1, -1),
      b3.reshape(1, -1))
